# baseline (speedup 1.0000x reference)
import functools

import jax
import jax.numpy as jnp
from jax import lax
from jax.experimental import pallas as pl
from jax.experimental.pallas import tpu as pltpu
from jax.experimental.pallas import tpu_sc as plsc

SEQ = 200
D = 128
CH = 100
NW = 32
NBUF = 4
NSB = 2
LANES = 16


def _body(x_hbm, tok_hbm, pos_hbm, out_hbm, pos_v,
          i0, i1, i2, i3, b0, b1, b2, b3, o0, o1,
          is0, is1, is2, is3, gs0, gs1, gs2, gs3, ss0, ss1):
    idx = [i0, i1, i2, i3]
    brows = [b0, b1, b2, b3]
    sbuf = [o0, o1]
    isem = [is0, is1, is2, is3]
    gsem = [gs0, gs1, gs2, gs3]
    ssem = [ss0, ss1]

    nchunk = x_hbm.shape[1]
    wid = lax.axis_index("s") * 2 + lax.axis_index("c")
    out_base = wid * (nchunk * SEQ)

    pltpu.sync_copy(pos_hbm, pos_v)

    def launch_idx(g, s):
        pltpu.async_copy(x_hbm.at[wid, g], idx[s], isem[s])

    def wait_idx(s):
        pltpu.make_async_copy(x_hbm.at[0, 0], idx[s], isem[s]).wait()

    def launch_gather(s):
        pltpu.async_copy(
            tok_hbm.at[idx[s].at[0]], brows[s].at[pl.ds(0, CH)], gsem[s])
        pltpu.async_copy(
            tok_hbm.at[idx[s].at[1]], brows[s].at[pl.ds(CH, CH)], gsem[s])

    def wait_gather(s):
        pltpu.make_async_copy(
            tok_hbm.at[pl.ds(0, CH)], brows[s].at[pl.ds(0, CH)],
            gsem[s]).wait()
        pltpu.make_async_copy(
            tok_hbm.at[pl.ds(0, CH)], brows[s].at[pl.ds(CH, CH)],
            gsem[s]).wait()

    def launch_store(g, s):
        pltpu.async_copy(
            sbuf[s], out_hbm.at[pl.ds(out_base + g * SEQ, SEQ)], ssem[s])

    def wait_store(s):
        pltpu.make_async_copy(
            sbuf[s], out_hbm.at[pl.ds(0, SEQ)], ssem[s]).wait()

    launch_idx(0, 0)
    launch_idx(1, 1)
    wait_idx(0)
    launch_gather(0)

    def grp_body(grp, _):
        g0 = grp * NBUF
        for b in range(NBUF):
            g = g0 + b
            s_next = (b + 1) % NBUF
            s_i = (b + 2) % NBUF
            s_s = b % NSB

            @pl.when(g + 2 < nchunk)
            def _():
                launch_idx(g + 2, s_i)

            @pl.when(g + 1 < nchunk)
            def _():
                wait_idx(s_next)
                launch_gather(s_next)

            wait_gather(b)

            @pl.when(g >= NSB)
            def _():
                wait_store(s_s)

            def row_body(r, _, b=b, s_s=s_s):
                for c in range(D // (2 * LANES)):
                    packed = brows[b][r, pl.ds(LANES * c, LANES)]
                    lo = lax.bitcast_convert_type(
                        lax.shift_left(packed, jnp.int32(16)), jnp.float32)
                    hi = lax.bitcast_convert_type(
                        lax.bitwise_and(packed, jnp.int32(-65536)),
                        jnp.float32)
                    sl_lo = pl.ds(2 * LANES * c, LANES)
                    sl_hi = pl.ds(2 * LANES * c + LANES, LANES)
                    sbuf[s_s][r, sl_lo] = lo + pos_v[r, sl_lo]
                    sbuf[s_s][r, sl_hi] = hi + pos_v[r, sl_hi]
                return 0

            lax.fori_loop(0, SEQ, row_body, 0)
            launch_store(g, s_s)
        return 0

    lax.fori_loop(0, nchunk // NBUF, grp_body, 0)

    for s in range(NSB):
        wait_store(s)


def kernel(x, token_table, pos_table):
    batch, seq = x.shape
    assert seq == SEQ
    ntok = batch * seq
    nchunk = ntok // (NW * SEQ)
    assert nchunk % NBUF == 0
    x_r = x.reshape(NW, nchunk, 2, CH).astype(jnp.int32)

    vocab = token_table.shape[0]
    tok_b = lax.bitcast_convert_type(
        token_table.astype(jnp.bfloat16)
        .reshape(vocab, D // 32, 2, 16)
        .transpose(0, 1, 3, 2),
        jnp.int32).reshape(vocab, D // 2)

    kern = functools.partial(
        pl.kernel,
        out_type=jax.ShapeDtypeStruct((ntok, D), jnp.float32),
        mesh=plsc.VectorSubcoreMesh(core_axis_name="c", subcore_axis_name="s"),
        compiler_params=pltpu.CompilerParams(use_tc_tiling_on_sc=False),
        scratch_types=(
            [pltpu.VMEM((SEQ, D), jnp.float32)]
            + [pltpu.VMEM((2, CH), jnp.int32)] * NBUF
            + [pltpu.VMEM((SEQ, D // 2), jnp.int32)] * NBUF
            + [pltpu.VMEM((SEQ, D), jnp.float32)] * NSB
            + [pltpu.SemaphoreType.DMA] * (2 * NBUF + NSB)
        ),
    )(_body)
    out = kern(x_r, tok_b, pos_table)
    return out.reshape(batch, seq, D)


# speedup vs baseline: 2.4865x; 2.4865x over previous
import functools

import jax
import jax.numpy as jnp
from jax import lax
from jax.experimental import pallas as pl
from jax.experimental.pallas import tpu as pltpu
from jax.experimental.pallas import tpu_sc as plsc

SEQ = 200
D = 128
CH = 100
NW = 32
NBUF = 4
LANES = 16


def _body(x_hbm, tok_hbm, pos_hbm, out_hbm, pos_v,
          i0, i1, i2, i3, r0, r1, r2, r3,
          is0, is1, is2, is3, gs0, gs1, gs2, gs3, ss0, ss1, ss2, ss3):
    idx = [i0, i1, i2, i3]
    rows = [r0, r1, r2, r3]
    isem = [is0, is1, is2, is3]
    gsem = [gs0, gs1, gs2, gs3]
    ssem = [ss0, ss1, ss2, ss3]

    nchunk = x_hbm.shape[1]
    wid = lax.axis_index("s") * 2 + lax.axis_index("c")
    out_base = wid * (nchunk * SEQ)

    pltpu.sync_copy(pos_hbm, pos_v)

    def launch_idx(g, s):
        pltpu.async_copy(x_hbm.at[wid, g], idx[s], isem[s])

    def wait_idx(s):
        pltpu.make_async_copy(x_hbm.at[0, 0], idx[s], isem[s]).wait()

    def launch_gather(s):
        pltpu.async_copy(
            tok_hbm.at[idx[s].at[0]], rows[s].at[pl.ds(0, CH)], gsem[s])
        pltpu.async_copy(
            tok_hbm.at[idx[s].at[1]], rows[s].at[pl.ds(CH, CH)], gsem[s])

    def wait_gather(s):
        pltpu.make_async_copy(
            tok_hbm.at[idx[s].at[0]], rows[s].at[pl.ds(0, CH)],
            gsem[s]).wait()
        pltpu.make_async_copy(
            tok_hbm.at[idx[s].at[1]], rows[s].at[pl.ds(CH, CH)],
            gsem[s]).wait()

    def launch_store(g, s):
        pltpu.async_copy(
            rows[s], out_hbm.at[pl.ds(out_base + g * SEQ, SEQ)], ssem[s])

    def wait_store(s):
        pltpu.make_async_copy(
            rows[s], out_hbm.at[pl.ds(0, SEQ)], ssem[s]).wait()

    launch_idx(0, 0)
    launch_idx(1, 1)
    wait_idx(0)
    launch_gather(0)

    def grp_body(grp, _):
        g0 = grp * NBUF
        for b in range(NBUF):
            g = g0 + b
            s_next = (b + 1) % NBUF
            s_i = (b + 2) % NBUF

            @pl.when(g + 2 < nchunk)
            def _():
                launch_idx(g + 2, s_i)

            @pl.when(g + 1 < nchunk)
            def _():
                wait_idx(s_next)

                @pl.when(g >= NBUF - 1)
                def _():
                    wait_store(s_next)

                launch_gather(s_next)

            wait_gather(b)

            def row_body(r, _, b=b):
                for c in range(D // LANES):
                    sl = pl.ds(c * LANES, LANES)
                    plsc.addupdate(rows[b].at[r, sl], pos_v[r, sl])
                return 0

            lax.fori_loop(0, SEQ, row_body, 0)
            launch_store(g, b)
        return 0

    lax.fori_loop(0, nchunk // NBUF, grp_body, 0)

    for s in range(NBUF):
        wait_store(s)


def kernel(x, token_table, pos_table):
    batch, seq = x.shape
    assert seq == SEQ
    ntok = batch * seq
    nchunk = ntok // (NW * SEQ)
    assert nchunk % NBUF == 0
    x_r = x.reshape(NW, nchunk, 2, CH).astype(jnp.int32)

    kern = functools.partial(
        pl.kernel,
        out_type=jax.ShapeDtypeStruct((ntok, D), jnp.float32),
        mesh=plsc.VectorSubcoreMesh(core_axis_name="c", subcore_axis_name="s"),
        scratch_types=(
            [pltpu.VMEM((SEQ, D), jnp.float32)]
            + [pltpu.VMEM((2, CH), jnp.int32)] * NBUF
            + [pltpu.VMEM((SEQ, D), jnp.float32)] * NBUF
            + [pltpu.SemaphoreType.DMA] * (3 * NBUF)
        ),
    )(_body)
    out = kern(x_r, token_table, pos_table)
    return out.reshape(batch, seq, D)
